# baseline (speedup 1.0000x reference)
import functools

import jax
import jax.numpy as jnp
from jax import lax
from jax.experimental import pallas as pl
from jax.experimental.pallas import tpu as pltpu
from jax.experimental.pallas import tpu_sc as plsc

N = 10000
E = 320000
D = 128
D_EDGE = 16

BN = 1000
BEDGE = 4000

NC, NS = 2, 16
NW = NC * NS
EPT = E // NW
BE = 40
NB = EPT // BE
NB2 = NB // 2
NPAD = 10240
RPT = NPAD // NS


def _node_dense(x, aw, ab, dw, db, ebw, ebb):
    def body(x_ref, aw_ref, ab_ref, dw_ref, db_ref, ebw_ref, ebb_ref,
             ax_ref, dx_ref, eb_ref):
        xb = x_ref[...]
        dn = (((1,), (1,)), ((), ()))
        ax_ref[...] = lax.dot_general(xb, aw_ref[...], dn,
                                      preferred_element_type=jnp.float32) + ab_ref[...]
        dx_ref[...] = lax.dot_general(xb, dw_ref[...], dn,
                                      preferred_element_type=jnp.float32) + db_ref[...]
        eb_ref[...] = lax.dot_general(xb, ebw_ref[...], dn,
                                      preferred_element_type=jnp.float32) + ebb_ref[...]

    grid = (N // BN,)
    return pl.pallas_call(
        body,
        grid=grid,
        in_specs=[
            pl.BlockSpec((BN, D), lambda i: (i, 0)),
            pl.BlockSpec((D, D), lambda i: (0, 0)),
            pl.BlockSpec((1, D), lambda i: (0, 0)),
            pl.BlockSpec((D, D), lambda i: (0, 0)),
            pl.BlockSpec((1, D), lambda i: (0, 0)),
            pl.BlockSpec((2 * D, D), lambda i: (0, 0)),
            pl.BlockSpec((1, 2 * D), lambda i: (0, 0)),
        ],
        out_specs=[
            pl.BlockSpec((BN, D), lambda i: (i, 0)),
            pl.BlockSpec((BN, D), lambda i: (i, 0)),
            pl.BlockSpec((BN, 2 * D), lambda i: (i, 0)),
        ],
        out_shape=[
            jax.ShapeDtypeStruct((N, D), jnp.float32),
            jax.ShapeDtypeStruct((N, D), jnp.float32),
            jax.ShapeDtypeStruct((N, 2 * D), jnp.float32),
        ],
    )(x, aw, ab, dw, db, ebw, ebb)


def _edge_proj(ea, cw, cb):
    def body(ea_ref, cw_ref, cb_ref, ce_ref):
        dn = (((1,), (1,)), ((), ()))
        ce_ref[...] = lax.dot_general(ea_ref[...], cw_ref[...], dn,
                                      preferred_element_type=jnp.float32) + cb_ref[...]

    return pl.pallas_call(
        body,
        grid=(E // BEDGE,),
        in_specs=[
            pl.BlockSpec((BEDGE, D_EDGE), lambda i: (i, 0)),
            pl.BlockSpec((D, D_EDGE), lambda i: (0, 0)),
            pl.BlockSpec((1, D), lambda i: (0, 0)),
        ],
        out_specs=pl.BlockSpec((BEDGE, D), lambda i: (i, 0)),
        out_shape=jax.ShapeDtypeStruct((E, D), jnp.float32),
    )(ea, cw, cb)


def _e_final(ea, wres, eij):
    def body(ea_ref, w_ref, r_ref, out_ref):
        dn = (((1,), (1,)), ((), ()))
        out_ref[...] = lax.dot_general(ea_ref[...], w_ref[...], dn,
                                       preferred_element_type=jnp.float32) + jnp.maximum(r_ref[...], 0.0)

    return pl.pallas_call(
        body,
        grid=(E // BEDGE,),
        in_specs=[
            pl.BlockSpec((BEDGE, D_EDGE), lambda i: (i, 0)),
            pl.BlockSpec((D, D_EDGE), lambda i: (0, 0)),
            pl.BlockSpec((BEDGE, D), lambda i: (i, 0)),
        ],
        out_specs=pl.BlockSpec((BEDGE, D), lambda i: (i, 0)),
        out_shape=jax.ShapeDtypeStruct((E, D), jnp.float32),
    )(ea, wres, eij)


def _x_final(x, ax, a0, a1):
    def body(x_ref, ax_ref, a0_ref, a1_ref, out_ref):
        out_ref[...] = x_ref[...] + jnp.maximum(
            ax_ref[...] + a0_ref[...] + a1_ref[...], 0.0)

    return pl.pallas_call(
        body,
        grid=(N // BN,),
        in_specs=[pl.BlockSpec((BN, D), lambda i: (i, 0))] * 4,
        out_specs=pl.BlockSpec((BN, D), lambda i: (i, 0)),
        out_shape=jax.ShapeDtypeStruct((N, D), jnp.float32),
    )(x, ax, a0, a1)


def _sc_edge(dx, eb, ce, row, col, w, zeros):
    mesh = plsc.VectorSubcoreMesh(core_axis_name="c", subcore_axis_name="s")

    buf_scratch = [
        pltpu.VMEM((BE,), jnp.int32),
        pltpu.VMEM((BE,), jnp.int32),
        pltpu.VMEM((BE,), jnp.float32),
        pltpu.VMEM((BE, D), jnp.float32),
        pltpu.VMEM((BE, 2 * D), jnp.float32),
        pltpu.VMEM((BE, D), jnp.float32),
        pltpu.SemaphoreType.DMA,
    ]

    @functools.partial(
        pl.kernel,
        out_type=(
            jax.ShapeDtypeStruct((E, D), jnp.float32),
            jax.ShapeDtypeStruct((NPAD, D), jnp.float32),
            jax.ShapeDtypeStruct((NPAD, D), jnp.float32),
        ),
        mesh=mesh,
        scratch_types=buf_scratch + buf_scratch + [
            pltpu.VMEM_SHARED((NPAD, D), jnp.float32),
        ],
    )
    def k(dx_hbm, eb_hbm, ce_hbm, row_hbm, col_hbm, w_hbm, z_hbm,
          eij_out, a0_out, a1_out, *rest):
        buf0 = rest[0:7]
        buf1 = rest[7:14]
        aggr_sh = rest[14]
        cid = lax.axis_index("c")
        sid = lax.axis_index("s")
        wid = cid * NS + sid
        ebase = wid * EPT

        pltpu.sync_copy(z_hbm.at[pl.ds(sid * RPT, RPT)],
                        aggr_sh.at[pl.ds(sid * RPT, RPT)])
        plsc.subcore_barrier()

        def idx_load(b, buf):
            base = ebase + b * BE
            pltpu.sync_copy(row_hbm.at[pl.ds(base, BE)], buf[0])
            pltpu.sync_copy(col_hbm.at[pl.ds(base, BE)], buf[1])
            pltpu.sync_copy(w_hbm.at[pl.ds(base, BE)], buf[2])

        def gather_copies(b, buf):
            base = ebase + b * BE
            semg = buf[6]
            return (pltpu.make_async_copy(dx_hbm.at[buf[0]], buf[3], semg),
                    pltpu.make_async_copy(eb_hbm.at[buf[1]], buf[4], semg),
                    pltpu.make_async_copy(ce_hbm.at[pl.ds(base, BE)], buf[5], semg))

        def start(copies):
            for cp in copies:
                cp.start()

        def wait(copies):
            for cp in copies:
                cp.wait()

        def compute(buf):
            w_v, d_v, eb_v, c_v = buf[2], buf[3], buf[4], buf[5]

            def edge_body(j, carry2):
                g = (j // 16) * 16
                w16 = w_v[pl.ds(g, 16)]
                jv = jnp.full((16, 1), j - g, dtype=jnp.int32)
                wj = lax.gather(
                    w16, jv,
                    lax.GatherDimensionNumbers(offset_dims=(),
                                               collapsed_slice_dims=(0,),
                                               start_index_map=(0,)),
                    (1,), mode=lax.GatherScatterMode.PROMISE_IN_BOUNDS)
                for kk in range(D // 16):
                    sl = pl.ds(kk * 16, 16)
                    dd = d_v[j, sl]
                    ee = eb_v[j, pl.ds(kk * 16, 16)]
                    bb = eb_v[j, pl.ds(D + kk * 16, 16)]
                    cc = c_v[j, sl]
                    eij = dd + ee + cc
                    a = jnp.exp(jnp.minimum(eij, -eij))
                    den = 1.0 + a
                    r = 1.4571 - 0.4571 * den
                    r = r * (2.0 - den * r)
                    r = r * (2.0 - den * r)
                    sig = jnp.where(eij > 0.0, r, a * r)
                    d_v[j, sl] = sig * bb * wj
                    c_v[j, sl] = eij
                return carry2

            lax.fori_loop(0, BE, edge_body, 0)

        def step(b, bufp, bufq):
            wait(gather_copies(b, bufp))

            @pl.when(b + 1 < NB)
            def _():
                idx_load(b + 1, bufq)
                start(gather_copies(b + 1, bufq))

            compute(bufp)
            base = ebase + b * BE
            pltpu.sync_copy(bufp[5], eij_out.at[pl.ds(base, BE)])
            pltpu.sync_copy(bufp[3], aggr_sh.at[bufp[0]], add=True)

        idx_load(0, buf0)
        start(gather_copies(0, buf0))

        def body2(i2, carry):
            step(i2 * 2, buf0, buf1)
            step(i2 * 2 + 1, buf1, buf0)
            return carry

        lax.fori_loop(0, NB2, body2, 0)
        plsc.subcore_barrier()

        @pl.when(cid == 0)
        def _():
            pltpu.sync_copy(aggr_sh.at[pl.ds(sid * RPT, RPT)],
                            a0_out.at[pl.ds(sid * RPT, RPT)])

        @pl.when(cid == 1)
        def _():
            pltpu.sync_copy(aggr_sh.at[pl.ds(sid * RPT, RPT)],
                            a1_out.at[pl.ds(sid * RPT, RPT)])

    return k(dx, eb, ce, row, col, w, zeros)


def kernel(x_in_node, edge_idx, edge_in_attr, edge_scalar_weights,
           A_w, A_b, B_w, B_b, C_w, C_b, D_w, D_b, E_w, E_b, Wres_e):
    ebw = jnp.concatenate([E_w, B_w], axis=0)
    ebb = jnp.concatenate([E_b, B_b])[None, :]
    ax, dx, eb = _node_dense(x_in_node, A_w, A_b[None, :], D_w, D_b[None, :],
                             ebw, ebb)
    ce = _edge_proj(edge_in_attr, C_w, C_b[None, :])
    row = edge_idx[0]
    col = edge_idx[1]
    zeros = jnp.zeros((NPAD, D), jnp.float32)
    eij, a0, a1 = _sc_edge(dx, eb, ce, row, col, edge_scalar_weights, zeros)
    e_final = _e_final(edge_in_attr, Wres_e, eij)
    x_final = _x_final(x_in_node, ax, a0, a1)
    return (x_final, e_final)


# speedup vs baseline: 1.1651x; 1.1651x over previous
import functools

import jax
import jax.numpy as jnp
from jax import lax
from jax.experimental import pallas as pl
from jax.experimental.pallas import tpu as pltpu
from jax.experimental.pallas import tpu_sc as plsc

N = 10000
E = 320000
D = 128
D_EDGE = 16

BN = 1000
BEDGE = 4000

NC, NS = 2, 16
NW = NC * NS
EPT = E // NW
BE = 40
NB = EPT // BE
NB2 = NB // 2
NPAD = 10240
RPT = NPAD // NS


def _node_dense(x, aw, ab, dw, db, ebw, ebb):
    def body(x_ref, aw_ref, ab_ref, dw_ref, db_ref, ebw_ref, ebb_ref,
             ax_ref, dx_ref, eb_ref):
        xb = x_ref[...]
        dn = (((1,), (1,)), ((), ()))
        ax_ref[...] = lax.dot_general(xb, aw_ref[...], dn,
                                      preferred_element_type=jnp.float32) + ab_ref[...]
        dx_ref[...] = lax.dot_general(xb, dw_ref[...], dn,
                                      preferred_element_type=jnp.float32) + db_ref[...]
        eb_ref[...] = lax.dot_general(xb, ebw_ref[...], dn,
                                      preferred_element_type=jnp.float32) + ebb_ref[...]

    grid = (N // BN,)
    return pl.pallas_call(
        body,
        grid=grid,
        in_specs=[
            pl.BlockSpec((BN, D), lambda i: (i, 0)),
            pl.BlockSpec((D, D), lambda i: (0, 0)),
            pl.BlockSpec((1, D), lambda i: (0, 0)),
            pl.BlockSpec((D, D), lambda i: (0, 0)),
            pl.BlockSpec((1, D), lambda i: (0, 0)),
            pl.BlockSpec((2 * D, D), lambda i: (0, 0)),
            pl.BlockSpec((1, 2 * D), lambda i: (0, 0)),
        ],
        out_specs=[
            pl.BlockSpec((BN, D), lambda i: (i, 0)),
            pl.BlockSpec((BN, D), lambda i: (i, 0)),
            pl.BlockSpec((BN, 2 * D), lambda i: (i, 0)),
        ],
        out_shape=[
            jax.ShapeDtypeStruct((N, D), jnp.float32),
            jax.ShapeDtypeStruct((N, D), jnp.float32),
            jax.ShapeDtypeStruct((N, 2 * D), jnp.float32),
        ],
    )(x, aw, ab, dw, db, ebw, ebb)


def _edge_proj(ea, cw, cb):
    def body(ea_ref, cw_ref, cb_ref, ce_ref):
        dn = (((1,), (1,)), ((), ()))
        ce_ref[...] = lax.dot_general(ea_ref[...], cw_ref[...], dn,
                                      preferred_element_type=jnp.float32) + cb_ref[...]

    return pl.pallas_call(
        body,
        grid=(E // BEDGE,),
        in_specs=[
            pl.BlockSpec((BEDGE, D_EDGE), lambda i: (i, 0)),
            pl.BlockSpec((D, D_EDGE), lambda i: (0, 0)),
            pl.BlockSpec((1, D), lambda i: (0, 0)),
        ],
        out_specs=pl.BlockSpec((BEDGE, D), lambda i: (i, 0)),
        out_shape=jax.ShapeDtypeStruct((E, D), jnp.float32),
    )(ea, cw, cb)


def _e_final(ea, wres, eij):
    def body(ea_ref, w_ref, r_ref, out_ref):
        dn = (((1,), (1,)), ((), ()))
        out_ref[...] = lax.dot_general(ea_ref[...], w_ref[...], dn,
                                       preferred_element_type=jnp.float32) + jnp.maximum(r_ref[...], 0.0)

    return pl.pallas_call(
        body,
        grid=(E // BEDGE,),
        in_specs=[
            pl.BlockSpec((BEDGE, D_EDGE), lambda i: (i, 0)),
            pl.BlockSpec((D, D_EDGE), lambda i: (0, 0)),
            pl.BlockSpec((BEDGE, D), lambda i: (i, 0)),
        ],
        out_specs=pl.BlockSpec((BEDGE, D), lambda i: (i, 0)),
        out_shape=jax.ShapeDtypeStruct((E, D), jnp.float32),
    )(ea, wres, eij)


def _x_final(x, ax, a0, a1):
    def body(x_ref, ax_ref, a0_ref, a1_ref, out_ref):
        out_ref[...] = x_ref[...] + jnp.maximum(
            ax_ref[...] + a0_ref[...] + a1_ref[...], 0.0)

    return pl.pallas_call(
        body,
        grid=(N // BN,),
        in_specs=[pl.BlockSpec((BN, D), lambda i: (i, 0))] * 4,
        out_specs=pl.BlockSpec((BN, D), lambda i: (i, 0)),
        out_shape=jax.ShapeDtypeStruct((N, D), jnp.float32),
    )(x, ax, a0, a1)


def _sc_edge(dx, eb, ce, row, col, w, zeros):
    mesh = plsc.VectorSubcoreMesh(core_axis_name="c", subcore_axis_name="s")

    buf_scratch = [
        pltpu.VMEM((BE,), jnp.int32),
        pltpu.VMEM((BE,), jnp.int32),
        pltpu.VMEM((BE,), jnp.float32),
        pltpu.VMEM((BE, D), jnp.float32),
        pltpu.VMEM((BE, 2 * D), jnp.float32),
        pltpu.VMEM((BE, D), jnp.float32),
        pltpu.SemaphoreType.DMA,
    ]

    @functools.partial(
        pl.kernel,
        out_type=(
            jax.ShapeDtypeStruct((E, D), jnp.float32),
            jax.ShapeDtypeStruct((NPAD, D), jnp.float32),
            jax.ShapeDtypeStruct((NPAD, D), jnp.float32),
        ),
        mesh=mesh,
        scratch_types=buf_scratch + buf_scratch + [
            pltpu.VMEM_SHARED((NPAD, D), jnp.float32),
        ],
    )
    def k(dx_hbm, eb_hbm, ce_hbm, row_hbm, col_hbm, w_hbm, z_hbm,
          eij_out, a0_out, a1_out, *rest):
        buf0 = rest[0:7]
        buf1 = rest[7:14]
        aggr_sh = rest[14]
        cid = lax.axis_index("c")
        sid = lax.axis_index("s")
        wid = cid * NS + sid
        ebase = wid * EPT

        pltpu.sync_copy(z_hbm.at[pl.ds(sid * RPT, RPT)],
                        aggr_sh.at[pl.ds(sid * RPT, RPT)])
        plsc.subcore_barrier()

        def idx_load(b, buf):
            base = ebase + b * BE
            pltpu.sync_copy(row_hbm.at[pl.ds(base, BE)], buf[0])
            pltpu.sync_copy(col_hbm.at[pl.ds(base, BE)], buf[1])
            pltpu.sync_copy(w_hbm.at[pl.ds(base, BE)], buf[2])

        def gather_copies(b, buf):
            base = ebase + b * BE
            semg = buf[6]
            return (pltpu.make_async_copy(dx_hbm.at[buf[0]], buf[3], semg),
                    pltpu.make_async_copy(eb_hbm.at[buf[1]], buf[4], semg),
                    pltpu.make_async_copy(ce_hbm.at[pl.ds(base, BE)], buf[5], semg))

        def start(copies):
            for cp in copies:
                cp.start()

        def wait(copies):
            for cp in copies:
                cp.wait()

        def compute(buf):
            w_v, d_v, eb_v, c_v = buf[2], buf[3], buf[4], buf[5]

            def edge_body(j, carry2):
                g = (j // 16) * 16
                w16 = w_v[pl.ds(g, 16)]
                jv = jnp.full((16, 1), j - g, dtype=jnp.int32)
                wj = lax.gather(
                    w16, jv,
                    lax.GatherDimensionNumbers(offset_dims=(),
                                               collapsed_slice_dims=(0,),
                                               start_index_map=(0,)),
                    (1,), mode=lax.GatherScatterMode.PROMISE_IN_BOUNDS)
                for kk in range(D // 16):
                    sl = pl.ds(kk * 16, 16)
                    dd = d_v[j, sl]
                    ee = eb_v[j, pl.ds(kk * 16, 16)]
                    bb = eb_v[j, pl.ds(D + kk * 16, 16)]
                    cc = c_v[j, sl]
                    eij = dd + ee + cc
                    sig = 1.0 / (1.0 + jnp.exp(-eij))
                    d_v[j, sl] = sig * bb * wj
                    c_v[j, sl] = eij
                return carry2

            lax.fori_loop(0, BE, edge_body, 0, unroll=4)

        def step(b, bufp, bufq):
            wait(gather_copies(b, bufp))

            @pl.when(b + 1 < NB)
            def _():
                idx_load(b + 1, bufq)
                start(gather_copies(b + 1, bufq))

            compute(bufp)
            base = ebase + b * BE
            pltpu.sync_copy(bufp[5], eij_out.at[pl.ds(base, BE)])
            pltpu.sync_copy(bufp[3], aggr_sh.at[bufp[0]], add=True)

        idx_load(0, buf0)
        start(gather_copies(0, buf0))

        def body2(i2, carry):
            step(i2 * 2, buf0, buf1)
            step(i2 * 2 + 1, buf1, buf0)
            return carry

        lax.fori_loop(0, NB2, body2, 0)
        plsc.subcore_barrier()

        @pl.when(cid == 0)
        def _():
            pltpu.sync_copy(aggr_sh.at[pl.ds(sid * RPT, RPT)],
                            a0_out.at[pl.ds(sid * RPT, RPT)])

        @pl.when(cid == 1)
        def _():
            pltpu.sync_copy(aggr_sh.at[pl.ds(sid * RPT, RPT)],
                            a1_out.at[pl.ds(sid * RPT, RPT)])

    return k(dx, eb, ce, row, col, w, zeros)


def kernel(x_in_node, edge_idx, edge_in_attr, edge_scalar_weights,
           A_w, A_b, B_w, B_b, C_w, C_b, D_w, D_b, E_w, E_b, Wres_e):
    ebw = jnp.concatenate([E_w, B_w], axis=0)
    ebb = jnp.concatenate([E_b, B_b])[None, :]
    ax, dx, eb = _node_dense(x_in_node, A_w, A_b[None, :], D_w, D_b[None, :],
                             ebw, ebb)
    ce = _edge_proj(edge_in_attr, C_w, C_b[None, :])
    row = edge_idx[0]
    col = edge_idx[1]
    zeros = jnp.zeros((NPAD, D), jnp.float32)
    eij, a0, a1 = _sc_edge(dx, eb, ce, row, col, edge_scalar_weights, zeros)
    e_final = _e_final(edge_in_attr, Wres_e, eij)
    x_final = _x_final(x_in_node, ax, a0, a1)
    return (x_final, e_final)


# speedup vs baseline: 1.2952x; 1.1117x over previous
import functools

import jax
import jax.numpy as jnp
from jax import lax
from jax.experimental import pallas as pl
from jax.experimental.pallas import tpu as pltpu
from jax.experimental.pallas import tpu_sc as plsc

N = 10000
E = 320000
D = 128
D_EDGE = 16

BN = 1000
BEDGE = 4000

NC, NS = 2, 16
NW = NC * NS
EPT = E // NW
BE = 40
NB = EPT // BE
NB2 = NB // 2
NPAD = 10240
RPT = NPAD // NS


def _node_dense(x, aw, ab, bw, bb, dw, db, ew, eb):
    def body(x_ref, aw_ref, ab_ref, bw_ref, bb_ref, dw_ref, db_ref,
             ew_ref, eb_ref, ax_ref, bx_ref, dx_ref, ex_ref):
        xb = x_ref[...]
        dn = (((1,), (1,)), ((), ()))
        ax_ref[...] = lax.dot_general(xb, aw_ref[...], dn,
                                      preferred_element_type=jnp.float32) + ab_ref[...]
        bx_ref[...] = lax.dot_general(xb, bw_ref[...], dn,
                                      preferred_element_type=jnp.float32) + bb_ref[...]
        dx_ref[...] = lax.dot_general(xb, dw_ref[...], dn,
                                      preferred_element_type=jnp.float32) + db_ref[...]
        ex_ref[...] = lax.dot_general(xb, ew_ref[...], dn,
                                      preferred_element_type=jnp.float32) + eb_ref[...]

    return pl.pallas_call(
        body,
        grid=(N // BN,),
        in_specs=[
            pl.BlockSpec((BN, D), lambda i: (i, 0)),
            pl.BlockSpec((D, D), lambda i: (0, 0)),
            pl.BlockSpec((1, D), lambda i: (0, 0)),
            pl.BlockSpec((D, D), lambda i: (0, 0)),
            pl.BlockSpec((1, D), lambda i: (0, 0)),
            pl.BlockSpec((D, D), lambda i: (0, 0)),
            pl.BlockSpec((1, D), lambda i: (0, 0)),
            pl.BlockSpec((D, D), lambda i: (0, 0)),
            pl.BlockSpec((1, D), lambda i: (0, 0)),
        ],
        out_specs=[pl.BlockSpec((BN, D), lambda i: (i, 0))] * 4,
        out_shape=[jax.ShapeDtypeStruct((N, D), jnp.float32)] * 4,
    )(x, aw, ab, bw, bb, dw, db, ew, eb)


def _edge_proj(ea, cw, cb):
    def body(ea_ref, cw_ref, cb_ref, ce_ref):
        dn = (((1,), (1,)), ((), ()))
        ce_ref[...] = lax.dot_general(ea_ref[...], cw_ref[...], dn,
                                      preferred_element_type=jnp.float32) + cb_ref[...]

    return pl.pallas_call(
        body,
        grid=(E // BEDGE,),
        in_specs=[
            pl.BlockSpec((BEDGE, D_EDGE), lambda i: (i, 0)),
            pl.BlockSpec((D, D_EDGE), lambda i: (0, 0)),
            pl.BlockSpec((1, D), lambda i: (0, 0)),
        ],
        out_specs=pl.BlockSpec((BEDGE, D), lambda i: (i, 0)),
        out_shape=jax.ShapeDtypeStruct((E, D), jnp.float32),
    )(ea, cw, cb)


def _e_final(ea, wres, eij):
    def body(ea_ref, w_ref, r_ref, out_ref):
        dn = (((1,), (1,)), ((), ()))
        out_ref[...] = lax.dot_general(ea_ref[...], w_ref[...], dn,
                                       preferred_element_type=jnp.float32) + jnp.maximum(r_ref[...], 0.0)

    return pl.pallas_call(
        body,
        grid=(E // BEDGE,),
        in_specs=[
            pl.BlockSpec((BEDGE, D_EDGE), lambda i: (i, 0)),
            pl.BlockSpec((D, D_EDGE), lambda i: (0, 0)),
            pl.BlockSpec((BEDGE, D), lambda i: (i, 0)),
        ],
        out_specs=pl.BlockSpec((BEDGE, D), lambda i: (i, 0)),
        out_shape=jax.ShapeDtypeStruct((E, D), jnp.float32),
    )(ea, wres, eij)


def _x_final(x, ax, a0, a1):
    def body(x_ref, ax_ref, a0_ref, a1_ref, out_ref):
        out_ref[...] = x_ref[...] + jnp.maximum(
            ax_ref[...] + a0_ref[...] + a1_ref[...], 0.0)

    return pl.pallas_call(
        body,
        grid=(N // BN,),
        in_specs=[pl.BlockSpec((BN, D), lambda i: (i, 0))] * 4,
        out_specs=pl.BlockSpec((BN, D), lambda i: (i, 0)),
        out_shape=jax.ShapeDtypeStruct((N, D), jnp.float32),
    )(x, ax, a0, a1)


def _sc_edge(dx, ex, bx, ce, row, col, w, zeros):
    mesh = plsc.VectorSubcoreMesh(core_axis_name="c", subcore_axis_name="s")

    buf_scratch = [
        pltpu.VMEM((BE,), jnp.int32),
        pltpu.VMEM((BE,), jnp.int32),
        pltpu.VMEM((BE,), jnp.float32),
        pltpu.VMEM((BE, D), jnp.float32),
        pltpu.VMEM((BE, D), jnp.float32),
        pltpu.SemaphoreType.DMA,
        pltpu.SemaphoreType.DMA,
    ]

    @functools.partial(
        pl.kernel,
        out_type=(
            jax.ShapeDtypeStruct((E, D), jnp.float32),
            jax.ShapeDtypeStruct((NPAD, D), jnp.float32),
            jax.ShapeDtypeStruct((NPAD, D), jnp.float32),
        ),
        mesh=mesh,
        scratch_types=buf_scratch + buf_scratch + [
            pltpu.VMEM((BE, D), jnp.float32),
            pltpu.VMEM_SHARED((NPAD, D), jnp.float32),
        ],
    )
    def k(dx_hbm, ex_hbm, bx_hbm, ce_hbm, row_hbm, col_hbm, w_hbm, z_hbm,
          eij_out, a0_out, a1_out, *rest):
        buf0 = rest[0:7]
        buf1 = rest[7:14]
        m_v = rest[14]
        aggr_sh = rest[15]
        cid = lax.axis_index("c")
        sid = lax.axis_index("s")
        wid = cid * NS + sid
        ebase = wid * EPT

        pltpu.sync_copy(z_hbm.at[pl.ds(sid * RPT, RPT)],
                        aggr_sh.at[pl.ds(sid * RPT, RPT)])
        plsc.subcore_barrier()

        def idx_load(b, buf):
            base = ebase + b * BE
            pltpu.sync_copy(row_hbm.at[pl.ds(base, BE)], buf[0])
            pltpu.sync_copy(col_hbm.at[pl.ds(base, BE)], buf[1])
            pltpu.sync_copy(w_hbm.at[pl.ds(base, BE)], buf[2])

        def ce_copy(b, buf):
            base = ebase + b * BE
            return pltpu.make_async_copy(ce_hbm.at[pl.ds(base, BE)], buf[3],
                                         buf[5])

        def gather_start(b, buf):
            pltpu.async_copy(dx_hbm.at[buf[0]], buf[3], buf[6], add=True)
            pltpu.async_copy(ex_hbm.at[buf[1]], buf[3], buf[6], add=True)
            pltpu.async_copy(bx_hbm.at[buf[1]], buf[4], buf[6])

        def gather_wait(b, buf):
            pltpu.make_async_copy(dx_hbm.at[buf[0]], buf[3], buf[6]).wait()
            pltpu.make_async_copy(ex_hbm.at[buf[1]], buf[3], buf[6]).wait()
            pltpu.make_async_copy(bx_hbm.at[buf[1]], buf[4], buf[6]).wait()

        def compute(buf):
            w_v, c_v, b_v = buf[2], buf[3], buf[4]

            def edge_body(j, carry2):
                g = (j // 16) * 16
                w16 = w_v[pl.ds(g, 16)]
                jv = jnp.full((16, 1), j - g, dtype=jnp.int32)
                wj = lax.gather(
                    w16, jv,
                    lax.GatherDimensionNumbers(offset_dims=(),
                                               collapsed_slice_dims=(0,),
                                               start_index_map=(0,)),
                    (1,), mode=lax.GatherScatterMode.PROMISE_IN_BOUNDS)
                for kk in range(D // 16):
                    sl = pl.ds(kk * 16, 16)
                    eij = c_v[j, sl]
                    bb = b_v[j, sl]
                    sig = 1.0 / (1.0 + jnp.exp(-eij))
                    m_v[j, sl] = sig * bb * wj
                return carry2

            lax.fori_loop(0, BE, edge_body, 0, unroll=4)

        def step(b, bufp, bufq):
            gather_wait(b, bufp)

            @pl.when(b + 1 < NB)
            def _():
                ce_copy(b + 1, bufq).start()
                idx_load(b + 1, bufq)
                ce_copy(b + 1, bufq).wait()
                gather_start(b + 1, bufq)

            compute(bufp)
            base = ebase + b * BE
            pltpu.sync_copy(bufp[3], eij_out.at[pl.ds(base, BE)])
            pltpu.sync_copy(m_v, aggr_sh.at[bufp[0]], add=True)

        idx_load(0, buf0)
        ce_copy(0, buf0).start()
        ce_copy(0, buf0).wait()
        gather_start(0, buf0)

        def body2(i2, carry):
            step(i2 * 2, buf0, buf1)
            step(i2 * 2 + 1, buf1, buf0)
            return carry

        lax.fori_loop(0, NB2, body2, 0)
        plsc.subcore_barrier()

        @pl.when(cid == 0)
        def _():
            pltpu.sync_copy(aggr_sh.at[pl.ds(sid * RPT, RPT)],
                            a0_out.at[pl.ds(sid * RPT, RPT)])

        @pl.when(cid == 1)
        def _():
            pltpu.sync_copy(aggr_sh.at[pl.ds(sid * RPT, RPT)],
                            a1_out.at[pl.ds(sid * RPT, RPT)])

    return k(dx, ex, bx, ce, row, col, w, zeros)


def kernel(x_in_node, edge_idx, edge_in_attr, edge_scalar_weights,
           A_w, A_b, B_w, B_b, C_w, C_b, D_w, D_b, E_w, E_b, Wres_e):
    ax, bxp, dxp, exp_ = _node_dense(x_in_node, A_w, A_b[None, :],
                                     B_w, B_b[None, :], D_w, D_b[None, :],
                                     E_w, E_b[None, :])
    ce = _edge_proj(edge_in_attr, C_w, C_b[None, :])
    row = edge_idx[0]
    col = edge_idx[1]
    zeros = jnp.zeros((NPAD, D), jnp.float32)
    eij, a0, a1 = _sc_edge(dxp, exp_, bxp, ce, row, col,
                           edge_scalar_weights, zeros)
    e_final = _e_final(edge_in_attr, Wres_e, eij)
    x_final = _x_final(x_in_node, ax, a0, a1)
    return (x_final, e_final)
